# baseline (speedup 1.0000x reference)
import functools

import jax
import jax.numpy as jnp
import numpy as np
from jax.experimental import pallas as pl
from jax.experimental.pallas import tpu as pltpu

SCALE = 1000000.0
LANES = 128
ROWS_PER_BLOCK = 512

_INTERPRET = False


def _helmert_basis(D):
    H = np.zeros((D - 1, D), dtype=np.float32)
    for i in range(D - 1):
        H[i, : i + 1] = 1.0 / (i + 1)
        H[i, i + 1] = -1.0
        H[i] *= np.sqrt((i + 1) / (i + 2))
    return H


def _fused_kernel(T1, S, x_ref, seg_ref, w_ref, v_ref, b_ref, out_ref):
    i = pl.program_id(0)
    nb = pl.num_programs(0)

    @pl.when(i == 0)
    def _init():
        out_ref[:] = jnp.zeros_like(out_ref)

    xb = x_ref[:]
    xw = jnp.dot(xb, w_ref[:], preferred_element_type=jnp.float32)
    rowsum = xw[:, T1 - 1 : T1]
    scale = SCALE / jnp.maximum(rowsum, 1e-8)
    ilr = xw * scale + b_ref[0, :][None, :]
    logx = jnp.dot(ilr, v_ref[:], preferred_element_type=jnp.float32)
    lane = jax.lax.broadcasted_iota(jnp.int32, logx.shape, 1)
    valid = lane < T1
    masked = jnp.where(valid, logx, -jnp.inf)
    m = jnp.max(masked, axis=1, keepdims=True)
    e = jnp.where(valid, jnp.exp(logx - m), 0.0)
    y = e / jnp.sum(e, axis=1, keepdims=True)

    seg = seg_ref[0]
    onehot = (jax.lax.broadcasted_iota(jnp.int32, (S, seg.shape[1]), 0) == seg)
    out_ref[:] += jnp.dot(onehot.astype(jnp.float32), y,
                          preferred_element_type=jnp.float32)

    @pl.when(i == nb - 1)
    def _finish():
        acc = out_ref[:]
        denom = jnp.maximum(jnp.sum(acc, axis=1, keepdims=True), 1e-8)
        out_ref[:] = acc / denom


def kernel(X_batch, batch_idx, W, b):
    N, G = X_batch.shape
    T = W.shape[1]
    T1 = T + 1
    S = 256

    R = ROWS_PER_BLOCK
    nb = N // R
    assert N % R == 0

    V = _helmert_basis(T1)
    V_pad = np.zeros((LANES, LANES), dtype=np.float32)
    V_pad[:T, :T1] = V
    V_pad = jnp.asarray(V_pad)
    W_pad = jnp.zeros((G, LANES), jnp.float32).at[:, :T].set(W)
    W_pad = W_pad.at[:, T].set(1.0)
    b_pad = jnp.zeros((1, LANES), jnp.float32).at[0, :T].set(b)
    seg3 = batch_idx.astype(jnp.int32).reshape(nb, 1, R)

    out = pl.pallas_call(
        functools.partial(_fused_kernel, T1, S),
        grid=(nb,),
        in_specs=[
            pl.BlockSpec((R, G), lambda i: (i, 0)),
            pl.BlockSpec((1, 1, R), lambda i: (i, 0, 0)),
            pl.BlockSpec((G, LANES), lambda i: (0, 0)),
            pl.BlockSpec((LANES, LANES), lambda i: (0, 0)),
            pl.BlockSpec((1, LANES), lambda i: (0, 0)),
        ],
        out_specs=pl.BlockSpec((S, LANES), lambda i: (0, 0)),
        out_shape=jax.ShapeDtypeStruct((S, LANES), jnp.float32),
        interpret=_INTERPRET,
    )(X_batch, seg3, W_pad, V_pad, b_pad)
    return out[:, :T1]


# speedup vs baseline: 1.0128x; 1.0128x over previous
import functools

import jax
import jax.numpy as jnp
import numpy as np
from jax.experimental import pallas as pl
from jax.experimental.pallas import tpu as pltpu

SCALE = 1000000.0
LANES = 128
ROWS_PER_BLOCK = 512

_INTERPRET = False


def _helmert_basis(D):
    H = np.zeros((D - 1, D), dtype=np.float32)
    for i in range(D - 1):
        H[i, : i + 1] = 1.0 / (i + 1)
        H[i, i + 1] = -1.0
        H[i] *= np.sqrt((i + 1) / (i + 2))
    return H


def _fused_kernel(T1, S, x_ref, seg_ref, w_ref, v_ref, b_ref, out_ref):
    i = pl.program_id(0)
    nb = pl.num_programs(0)

    @pl.when(i == 0)
    def _init():
        out_ref[:] = jnp.zeros_like(out_ref)

    xb = x_ref[:]
    rowsum = jnp.sum(xb, axis=1, keepdims=True)
    xn = xb * (SCALE / jnp.maximum(rowsum, 1e-8))
    ilr = jnp.dot(xn, w_ref[:], preferred_element_type=jnp.float32)
    ilr = ilr + b_ref[0, :][None, :]
    logx = jnp.dot(ilr, v_ref[:], preferred_element_type=jnp.float32)
    lane = jax.lax.broadcasted_iota(jnp.int32, logx.shape, 1)
    valid = lane < T1
    masked = jnp.where(valid, logx, -jnp.inf)
    m = jnp.max(masked, axis=1, keepdims=True)
    e = jnp.where(valid, jnp.exp(logx - m), 0.0)
    y = e / jnp.sum(e, axis=1, keepdims=True)

    seg = seg_ref[0]
    onehot = (jax.lax.broadcasted_iota(jnp.int32, (S, seg.shape[1]), 0) == seg)
    out_ref[:] += jnp.dot(onehot.astype(jnp.float32), y,
                          preferred_element_type=jnp.float32)

    @pl.when(i == nb - 1)
    def _finish():
        acc = out_ref[:]
        denom = jnp.maximum(jnp.sum(acc, axis=1, keepdims=True), 1e-8)
        out_ref[:] = acc / denom


def kernel(X_batch, batch_idx, W, b):
    N, G = X_batch.shape
    T = W.shape[1]
    T1 = T + 1
    S = 256

    R = ROWS_PER_BLOCK
    nb = N // R
    assert N % R == 0

    V = _helmert_basis(T1)
    V_pad = np.zeros((LANES, LANES), dtype=np.float32)
    V_pad[:T, :T1] = V
    V_pad = jnp.asarray(V_pad)
    W_pad = jnp.zeros((G, LANES), jnp.float32).at[:, :T].set(W)
    b_pad = jnp.zeros((1, LANES), jnp.float32).at[0, :T].set(b)
    seg3 = batch_idx.astype(jnp.int32).reshape(nb, 1, R)

    out = pl.pallas_call(
        functools.partial(_fused_kernel, T1, S),
        grid=(nb,),
        in_specs=[
            pl.BlockSpec((R, G), lambda i: (i, 0)),
            pl.BlockSpec((1, 1, R), lambda i: (i, 0, 0)),
            pl.BlockSpec((G, LANES), lambda i: (0, 0)),
            pl.BlockSpec((LANES, LANES), lambda i: (0, 0)),
            pl.BlockSpec((1, LANES), lambda i: (0, 0)),
        ],
        out_specs=pl.BlockSpec((S, LANES), lambda i: (0, 0)),
        out_shape=jax.ShapeDtypeStruct((S, LANES), jnp.float32),
        interpret=_INTERPRET,
    )(X_batch, seg3, W_pad, V_pad, b_pad)
    return out[:, :T1]


# speedup vs baseline: 1.2598x; 1.2439x over previous
import functools

import jax
import jax.numpy as jnp
import numpy as np
from jax.experimental import pallas as pl
from jax.experimental.pallas import tpu as pltpu

SCALE = 1000000.0
LANES = 128
ROWS_PER_BLOCK = 1024

_INTERPRET = False


def _helmert_basis(D):
    H = np.zeros((D - 1, D), dtype=np.float32)
    for i in range(D - 1):
        H[i, : i + 1] = 1.0 / (i + 1)
        H[i, i + 1] = -1.0
        H[i] *= np.sqrt((i + 1) / (i + 2))
    return H


def _fused_kernel(T1, S, x_ref, seg_ref, w_ref, v_ref, b_ref, out_ref):
    i = pl.program_id(0)
    nb = pl.num_programs(0)

    @pl.when(i == 0)
    def _init():
        out_ref[:] = jnp.zeros_like(out_ref)

    xb = x_ref[:]
    rowsum = jnp.sum(xb, axis=1, keepdims=True)
    xn = xb * (SCALE / jnp.maximum(rowsum, 1e-8))
    ilr = jnp.dot(xn, w_ref[:], preferred_element_type=jnp.float32)
    ilr = ilr + b_ref[0, :][None, :]
    logx = jnp.dot(ilr, v_ref[:], preferred_element_type=jnp.float32)
    lane = jax.lax.broadcasted_iota(jnp.int32, logx.shape, 1)
    valid = lane < T1
    masked = jnp.where(valid, logx, -jnp.inf)
    m = jnp.max(masked, axis=1, keepdims=True)
    e = jnp.where(valid, jnp.exp(logx - m), 0.0)
    y = e / jnp.sum(e, axis=1, keepdims=True)

    seg = seg_ref[0]
    onehot = (jax.lax.broadcasted_iota(jnp.int32, (S, seg.shape[1]), 0) == seg)
    out_ref[:] += jnp.dot(onehot.astype(jnp.float32), y,
                          preferred_element_type=jnp.float32)

    @pl.when(i == nb - 1)
    def _finish():
        acc = out_ref[:]
        denom = jnp.maximum(jnp.sum(acc, axis=1, keepdims=True), 1e-8)
        out_ref[:] = acc / denom


def kernel(X_batch, batch_idx, W, b):
    N, G = X_batch.shape
    T = W.shape[1]
    T1 = T + 1
    S = 256

    R = ROWS_PER_BLOCK
    nb = N // R
    assert N % R == 0

    V = _helmert_basis(T1)
    V_pad = np.zeros((LANES, LANES), dtype=np.float32)
    V_pad[:T, :T1] = V
    V_pad = jnp.asarray(V_pad)
    W_pad = jnp.zeros((G, LANES), jnp.float32).at[:, :T].set(W)
    b_pad = jnp.zeros((1, LANES), jnp.float32).at[0, :T].set(b)
    seg3 = batch_idx.astype(jnp.int32).reshape(nb, 1, R)

    out = pl.pallas_call(
        functools.partial(_fused_kernel, T1, S),
        grid=(nb,),
        in_specs=[
            pl.BlockSpec((R, G), lambda i: (i, 0)),
            pl.BlockSpec((1, 1, R), lambda i: (i, 0, 0)),
            pl.BlockSpec((G, LANES), lambda i: (0, 0)),
            pl.BlockSpec((LANES, LANES), lambda i: (0, 0)),
            pl.BlockSpec((1, LANES), lambda i: (0, 0)),
        ],
        out_specs=pl.BlockSpec((S, LANES), lambda i: (0, 0)),
        out_shape=jax.ShapeDtypeStruct((S, LANES), jnp.float32),
        interpret=_INTERPRET,
    )(X_batch, seg3, W_pad, V_pad, b_pad)
    return out[:, :T1]


# speedup vs baseline: 1.4554x; 1.1552x over previous
import functools

import jax
import jax.numpy as jnp
import numpy as np
from jax.experimental import pallas as pl
from jax.experimental.pallas import tpu as pltpu

SCALE = 1000000.0
LANES = 128
ROWS_PER_BLOCK = 2048

_INTERPRET = False


def _helmert_basis(D):
    H = np.zeros((D - 1, D), dtype=np.float32)
    for i in range(D - 1):
        H[i, : i + 1] = 1.0 / (i + 1)
        H[i, i + 1] = -1.0
        H[i] *= np.sqrt((i + 1) / (i + 2))
    return H


def _fused_kernel(T1, S, x_ref, seg_ref, w_ref, v_ref, b_ref, out_ref):
    i = pl.program_id(0)
    nb = pl.num_programs(0)

    @pl.when(i == 0)
    def _init():
        out_ref[:] = jnp.zeros_like(out_ref)

    xb = x_ref[:]
    rowsum = jnp.sum(xb, axis=1, keepdims=True)
    xn = xb * (SCALE / jnp.maximum(rowsum, 1e-8))
    ilr = jnp.dot(xn, w_ref[:], preferred_element_type=jnp.float32)
    ilr = ilr + b_ref[0, :][None, :]
    logx = jnp.dot(ilr, v_ref[:], preferred_element_type=jnp.float32)
    lane = jax.lax.broadcasted_iota(jnp.int32, logx.shape, 1)
    valid = lane < T1
    masked = jnp.where(valid, logx, -jnp.inf)
    m = jnp.max(masked, axis=1, keepdims=True)
    e = jnp.where(valid, jnp.exp(logx - m), 0.0)
    y = e / jnp.sum(e, axis=1, keepdims=True)

    seg = seg_ref[0]
    onehot = (jax.lax.broadcasted_iota(jnp.int32, (S, seg.shape[1]), 0) == seg)
    out_ref[:] += jnp.dot(onehot.astype(jnp.float32), y,
                          preferred_element_type=jnp.float32)

    @pl.when(i == nb - 1)
    def _finish():
        acc = out_ref[:]
        denom = jnp.maximum(jnp.sum(acc, axis=1, keepdims=True), 1e-8)
        out_ref[:] = acc / denom


def kernel(X_batch, batch_idx, W, b):
    N, G = X_batch.shape
    T = W.shape[1]
    T1 = T + 1
    S = 256

    R = ROWS_PER_BLOCK
    nb = N // R
    assert N % R == 0

    V = _helmert_basis(T1)
    V_pad = np.zeros((LANES, LANES), dtype=np.float32)
    V_pad[:T, :T1] = V
    V_pad = jnp.asarray(V_pad)
    W_pad = jnp.zeros((G, LANES), jnp.float32).at[:, :T].set(W)
    b_pad = jnp.zeros((1, LANES), jnp.float32).at[0, :T].set(b)
    seg3 = batch_idx.astype(jnp.int32).reshape(nb, 1, R)

    out = pl.pallas_call(
        functools.partial(_fused_kernel, T1, S),
        grid=(nb,),
        in_specs=[
            pl.BlockSpec((R, G), lambda i: (i, 0)),
            pl.BlockSpec((1, 1, R), lambda i: (i, 0, 0)),
            pl.BlockSpec((G, LANES), lambda i: (0, 0)),
            pl.BlockSpec((LANES, LANES), lambda i: (0, 0)),
            pl.BlockSpec((1, LANES), lambda i: (0, 0)),
        ],
        out_specs=pl.BlockSpec((S, LANES), lambda i: (0, 0)),
        out_shape=jax.ShapeDtypeStruct((S, LANES), jnp.float32),
        interpret=_INTERPRET,
    )(X_batch, seg3, W_pad, V_pad, b_pad)
    return out[:, :T1]
